# Initial kernel scaffold; baseline (speedup 1.0000x reference)
import jax
import jax.numpy as jnp
from jax.experimental import pallas as pl


def kernel(x, pe):
    raise NotImplementedError("write your pallas kernel here")


# speedup vs baseline: 1.6171x; 1.6171x over previous
import functools
import jax
import jax.numpy as jnp
from jax import lax
from jax.experimental import pallas as pl
from jax.experimental.pallas import tpu as pltpu
from jax.experimental.pallas import tpu_sc as plsc

_NUM_CORES = 2
_NUM_SUBCORES = 16
_NW = _NUM_CORES * _NUM_SUBCORES

_B = 16384
_D = 2048
_BPW = _B // _NW
_C = 16
_NCHUNK = _BPW // _C

_mesh = plsc.VectorSubcoreMesh(core_axis_name="c", subcore_axis_name="s")


@functools.partial(
    pl.kernel,
    out_type=jax.ShapeDtypeStruct((_B, _D), jnp.float32),
    mesh=_mesh,
    scratch_types=[
        pltpu.VMEM((_BPW,), jnp.int32),
        pltpu.VMEM((_C, _D), jnp.float32),
        pltpu.SemaphoreType.DMA,
    ],
)
def _gather(table_hbm, idx_hbm, out_hbm, idx_v, rows_v, gsem):
    wid = lax.axis_index("s") * _NUM_CORES + lax.axis_index("c")
    base = wid * _BPW
    pltpu.sync_copy(idx_hbm.at[pl.ds(base, _BPW)], idx_v)

    def chunk_body(g, carry):
        pltpu.async_copy(
            table_hbm.at[idx_v.at[pl.ds(g * _C, _C)]], rows_v, gsem
        ).wait()
        pltpu.sync_copy(rows_v, out_hbm.at[pl.ds(base + g * _C, _C)])
        return carry

    lax.fori_loop(0, _NCHUNK, chunk_body, 0)


def kernel(x, pe):
    xf = x.reshape(-1).astype(jnp.int32)
    out = _gather(pe, xf)
    return out.reshape(x.shape[0], x.shape[1], pe.shape[1])


# speedup vs baseline: 1.9531x; 1.2078x over previous
import functools
import jax
import jax.numpy as jnp
from jax import lax
from jax.experimental import pallas as pl
from jax.experimental.pallas import tpu as pltpu
from jax.experimental.pallas import tpu_sc as plsc

_NUM_CORES = 2
_NUM_SUBCORES = 16
_NW = _NUM_CORES * _NUM_SUBCORES

_B = 16384
_D = 2048
_BPW = _B // _NW
_C = 16
_NCHUNK = _BPW // _C

_mesh = plsc.VectorSubcoreMesh(core_axis_name="c", subcore_axis_name="s")


@functools.partial(
    pl.kernel,
    out_type=jax.ShapeDtypeStruct((_B, _D), jnp.float32),
    mesh=_mesh,
    scratch_types=[
        pltpu.VMEM((_BPW,), jnp.int32),
        pltpu.VMEM((_C, _D), jnp.float32),
        pltpu.VMEM((_C, _D), jnp.float32),
        pltpu.SemaphoreType.DMA,
        pltpu.SemaphoreType.DMA,
    ],
)
def _gather(table_hbm, idx_hbm, out_hbm, idx_v, rows0, rows1, gsem, osem):
    wid = lax.axis_index("s") * _NUM_CORES + lax.axis_index("c")
    base = wid * _BPW
    pltpu.sync_copy(idx_hbm.at[pl.ds(base, _BPW)], idx_v)

    bufs = (rows0, rows1)

    def start_gather(g, buf):
        pltpu.async_copy(table_hbm.at[idx_v.at[pl.ds(g * _C, _C)]], buf, gsem)

    def drain_gather(buf):
        pltpu.make_async_copy(table_hbm.at[pl.ds(0, _C)], buf, gsem).wait()

    def start_ocopy(g, buf):
        pltpu.async_copy(buf, out_hbm.at[pl.ds(base + g * _C, _C)], osem)

    def drain_ocopy(buf):
        pltpu.make_async_copy(buf, out_hbm.at[pl.ds(base, _C)], osem).wait()

    start_gather(0, bufs[0])

    @pl.loop(0, _NCHUNK, step=2)
    def _body(g0):
        for b in range(2):
            g = g0 + b
            buf = bufs[b]
            other = bufs[1 - b]

            @pl.when(g >= 1)
            def _():
                drain_ocopy(other)

            @pl.when(g + 1 < _NCHUNK)
            def _():
                start_gather(g + 1, other)

            drain_gather(buf)
            start_ocopy(g, buf)

    drain_ocopy(bufs[(_NCHUNK - 1) % 2])


def kernel(x, pe):
    xf = x.reshape(-1).astype(jnp.int32)
    out = _gather(pe, xf)
    return out.reshape(x.shape[0], x.shape[1], pe.shape[1])
